# baseline (speedup 1.0000x reference)
import functools

import jax
import jax.numpy as jnp
from jax import lax
from jax.experimental import pallas as pl
from jax.experimental.pallas import tpu as pltpu
from jax.experimental.pallas import tpu_sc as plsc

B, C, H, W = 64, 192, 56, 56
D = H * W
NROWS = B * C
NC, NS = 2, 16
NW = NC * NS
ROWS_PER_W = NROWS // NW
CHUNK = 16
NCHUNK = ROWS_PER_W // CHUNK
NBUF = 2


def _make_sc_gather():
    mesh = plsc.VectorSubcoreMesh(core_axis_name="c", subcore_axis_name="s")

    @functools.partial(
        pl.kernel,
        mesh=mesh,
        compiler_params=pltpu.CompilerParams(use_tc_tiling_on_sc=False),
        out_type=jax.ShapeDtypeStruct((NROWS, D), jnp.float32),
        scratch_types=[
            pltpu.VMEM((NCHUNK, CHUNK), jnp.int32),
            pltpu.VMEM((NBUF, CHUNK, D), jnp.float32),
            pltpu.SemaphoreType.DMA,
            pltpu.SemaphoreType.DMA,
            pltpu.SemaphoreType.DMA,
            pltpu.SemaphoreType.DMA,
        ],
    )
    def kern(x_hbm, idx_hbm, out_hbm, idx_v, buf, g0, g1, s0, s1):
        wid = lax.axis_index("s") * NC + lax.axis_index("c")
        base = wid * ROWS_PER_W
        pltpu.sync_copy(idx_hbm.at[wid], idx_v)
        gsems = (g0, g1)
        ssems = (s0, s1)

        def start_gather(ch, b):
            pltpu.async_copy(x_hbm.at[idx_v.at[ch]], buf.at[b], gsems[b])

        def wait_gather(b):
            pltpu.make_async_copy(x_hbm.at[pl.ds(0, CHUNK)], buf.at[b],
                                  gsems[b]).wait()

        def start_scatter(ch, b):
            pltpu.async_copy(buf.at[b], out_hbm.at[pl.ds(base + ch * CHUNK, CHUNK)],
                             ssems[b])

        def wait_scatter(ch, b):
            pltpu.make_async_copy(buf.at[b],
                                  out_hbm.at[pl.ds(base + ch * CHUNK, CHUNK)],
                                  ssems[b]).wait()

        start_gather(0, 0)
        start_gather(1, 1)

        def body(i, carry):
            chunks = [i * NBUF + b for b in range(NBUF)]
            for b in range(NBUF):
                wait_gather(b)
                start_scatter(chunks[b], b)
            for b in range(NBUF):
                wait_scatter(chunks[b], b)

                @pl.when(chunks[b] + NBUF < NCHUNK)
                def _():
                    start_gather(chunks[b] + NBUF, b)

            return carry

        lax.fori_loop(0, NCHUNK // NBUF, body, 0)

    return kern


_sc_gather = _make_sc_gather()


@jax.jit
def kernel(x, permutation):
    x2d = x.reshape(NROWS, D)
    rows = (jnp.arange(B, dtype=jnp.int32)[:, None] * C
            + permutation.astype(jnp.int32)[None, :])
    idx = rows.reshape(NW, NCHUNK, CHUNK)
    out = _sc_gather(x2d, idx)
    z = out.reshape(B, C, H, W)
    ldj = jnp.zeros((B,), dtype=x.dtype)
    return (z, ldj)


# speedup vs baseline: 1.5320x; 1.5320x over previous
import functools

import jax
import jax.numpy as jnp
from jax import lax
from jax.experimental import pallas as pl
from jax.experimental.pallas import tpu as pltpu
from jax.experimental.pallas import tpu_sc as plsc

B, C, H, W = 64, 192, 56, 56
NC, NS = 2, 16
NW = NC * NS
BCHUNK = 8
NBC = B // BCHUNK
NITEMS = C * NBC
ITEMS_PER_W = NITEMS // NW
NBUF = 2


def _make_sc_permute():
    mesh = plsc.VectorSubcoreMesh(core_axis_name="c", subcore_axis_name="s")

    @functools.partial(
        pl.kernel,
        mesh=mesh,
        out_type=jax.ShapeDtypeStruct((B, C, H, W), jnp.float32),
        scratch_types=[
            pltpu.VMEM((C + 16,), jnp.int32),
            pltpu.VMEM((NBUF, BCHUNK, 1, H, W), jnp.float32),
            pltpu.SemaphoreType.DMA,
            pltpu.SemaphoreType.DMA,
            pltpu.SemaphoreType.DMA,
            pltpu.SemaphoreType.DMA,
        ],
    )
    def kern(x_hbm, perm_hbm, out_hbm, perm_v, buf, g0, g1, s0, s1):
        wid = lax.axis_index("s") * NC + lax.axis_index("c")
        base = wid * ITEMS_PER_W
        pltpu.sync_copy(perm_hbm, perm_v.at[pl.ds(0, C)])
        gsems = (g0, g1)
        ssems = (s0, s1)

        def src_of(c):
            return perm_v[pl.ds(c, 16)][0]

        def start_gather(item, b):
            c = item // NBC
            b0 = (item % NBC) * BCHUNK
            src = src_of(c)
            pltpu.async_copy(
                x_hbm.at[pl.ds(b0, BCHUNK), pl.ds(src, 1)], buf.at[b], gsems[b])

        def wait_gather(b):
            pltpu.make_async_copy(
                x_hbm.at[pl.ds(0, BCHUNK), pl.ds(0, 1)], buf.at[b],
                gsems[b]).wait()

        def out_slice(item):
            c = item // NBC
            b0 = (item % NBC) * BCHUNK
            return out_hbm.at[pl.ds(b0, BCHUNK), pl.ds(c, 1)]

        def start_scatter(item, b):
            pltpu.async_copy(buf.at[b], out_slice(item), ssems[b])

        def wait_scatter(item, b):
            pltpu.make_async_copy(buf.at[b], out_slice(item), ssems[b]).wait()

        start_gather(base + 0, 0)
        start_gather(base + 1, 1)

        def body(i, carry):
            items = [base + i * NBUF + b for b in range(NBUF)]
            for b in range(NBUF):
                wait_gather(b)
                start_scatter(items[b], b)
            for b in range(NBUF):
                wait_scatter(items[b], b)

                @pl.when(i * NBUF + b + NBUF < ITEMS_PER_W)
                def _():
                    start_gather(items[b] + NBUF, b)

            return carry

        lax.fori_loop(0, ITEMS_PER_W // NBUF, body, 0)

    return kern


_sc_permute = _make_sc_permute()


@jax.jit
def kernel(x, permutation):
    z = _sc_permute(x, permutation.astype(jnp.int32))
    ldj = jnp.zeros((B,), dtype=x.dtype)
    return (z, ldj)


# speedup vs baseline: 1.5402x; 1.0053x over previous
import functools

import jax
import jax.numpy as jnp
from jax import lax
from jax.experimental import pallas as pl
from jax.experimental.pallas import tpu as pltpu
from jax.experimental.pallas import tpu_sc as plsc

B, C, H, W = 64, 192, 56, 56
NC, NS = 2, 16
NW = NC * NS
BCHUNK = 4
NBC = B // BCHUNK
NITEMS = C * NBC
ITEMS_PER_W = NITEMS // NW
NBUF = 4


def _make_sc_permute():
    mesh = plsc.VectorSubcoreMesh(core_axis_name="c", subcore_axis_name="s")

    @functools.partial(
        pl.kernel,
        mesh=mesh,
        out_type=jax.ShapeDtypeStruct((B, C, H, W), jnp.float32),
        scratch_types=[
            pltpu.VMEM((C + 16,), jnp.int32),
            pltpu.VMEM((NBUF, BCHUNK, 1, H, W), jnp.float32),
            pltpu.SemaphoreType.DMA,
            pltpu.SemaphoreType.DMA,
            pltpu.SemaphoreType.DMA,
            pltpu.SemaphoreType.DMA,
            pltpu.SemaphoreType.DMA,
            pltpu.SemaphoreType.DMA,
            pltpu.SemaphoreType.DMA,
            pltpu.SemaphoreType.DMA,
        ],
    )
    def kern(x_hbm, perm_hbm, out_hbm, perm_v, buf,
             g0, g1, g2, g3, s0, s1, s2, s3):
        wid = lax.axis_index("s") * NC + lax.axis_index("c")
        base = wid * ITEMS_PER_W
        pltpu.sync_copy(perm_hbm, perm_v.at[pl.ds(0, C)])
        gsems = (g0, g1, g2, g3)
        ssems = (s0, s1, s2, s3)

        def src_of(c):
            return perm_v[pl.ds(c, 16)][0]

        def start_gather(item, b):
            c = item // NBC
            b0 = (item % NBC) * BCHUNK
            src = src_of(c)
            pltpu.async_copy(
                x_hbm.at[pl.ds(b0, BCHUNK), pl.ds(src, 1)], buf.at[b], gsems[b])

        def wait_gather(b):
            pltpu.make_async_copy(
                x_hbm.at[pl.ds(0, BCHUNK), pl.ds(0, 1)], buf.at[b],
                gsems[b]).wait()

        def out_slice(item):
            c = item // NBC
            b0 = (item % NBC) * BCHUNK
            return out_hbm.at[pl.ds(b0, BCHUNK), pl.ds(c, 1)]

        def start_scatter(item, b):
            pltpu.async_copy(buf.at[b], out_slice(item), ssems[b])

        def wait_scatter(item, b):
            pltpu.make_async_copy(buf.at[b], out_slice(item), ssems[b]).wait()

        for b in range(NBUF):
            start_gather(base + b, b)

        def body(i, carry):
            items = [base + i * NBUF + b for b in range(NBUF)]
            for b in range(NBUF):
                wait_gather(b)
                start_scatter(items[b], b)
            for b in range(NBUF):
                wait_scatter(items[b], b)

                @pl.when(i * NBUF + b + NBUF < ITEMS_PER_W)
                def _():
                    start_gather(items[b] + NBUF, b)

            return carry

        lax.fori_loop(0, ITEMS_PER_W // NBUF, body, 0)

    return kern


_sc_permute = _make_sc_permute()


@jax.jit
def kernel(x, permutation):
    z = _sc_permute(x, permutation.astype(jnp.int32))
    ldj = jnp.zeros((B,), dtype=x.dtype)
    return (z, ldj)


# speedup vs baseline: 8.4557x; 5.4902x over previous
import jax
import jax.numpy as jnp
from jax.experimental import pallas as pl
from jax.experimental.pallas import tpu as pltpu

B, C, H, W = 64, 192, 56, 56
NPIX = B * H * W
PBLK = 2048
NGRID = NPIX // PBLK


def _permute_block(x_ref, m_ref, o_ref):
    o_ref[...] = jnp.dot(x_ref[...], m_ref[...],
                         preferred_element_type=jnp.float32)


def _lane_permute(x2, m):
    return pl.pallas_call(
        _permute_block,
        grid=(NGRID,),
        in_specs=[
            pl.BlockSpec((PBLK, C), lambda i: (i, 0)),
            pl.BlockSpec((C, C), lambda i: (0, 0)),
        ],
        out_specs=pl.BlockSpec((PBLK, C), lambda i: (i, 0)),
        out_shape=jax.ShapeDtypeStruct((NPIX, C), jnp.float32),
        compiler_params=pltpu.CompilerParams(
            dimension_semantics=("arbitrary",),
        ),
    )(x2, m)


@jax.jit
def kernel(x, permutation):
    x2 = x.transpose(0, 2, 3, 1).reshape(NPIX, C)
    m = (permutation[None, :] == jnp.arange(C, dtype=permutation.dtype)[:, None]
         ).astype(jnp.float32)
    out2 = _lane_permute(x2, m)
    z = out2.reshape(B, H, W, C).transpose(0, 3, 1, 2)
    ldj = jnp.zeros((B,), dtype=x.dtype)
    return (z, ldj)
